# baseline (speedup 1.0000x reference)
import functools

import jax
import jax.numpy as jnp
from jax import lax
from jax.experimental import pallas as pl
from jax.experimental.pallas import tpu as pltpu
from jax.experimental.pallas import tpu_sc as plsc

_N_DIMS = 64
_SEQ_LEN = 200
_BATCH = 4096
_NW = 32
_BC = 128
_NBC = _BATCH // _BC
_NTASK = _SEQ_LEN * _NBC
_TPW = _NTASK // _NW

_mesh = plsc.VectorSubcoreMesh(core_axis_name="c", subcore_axis_name="s")


@functools.partial(
    pl.kernel,
    mesh=_mesh,
    out_type=jax.ShapeDtypeStruct((_SEQ_LEN, _N_DIMS, _BATCH), jnp.float32),
    scratch_types=[
        pltpu.VMEM((2, _BC), jnp.int32),
        pltpu.VMEM((2, _BC), jnp.int32),
        pltpu.VMEM((2, _BC, _N_DIMS), jnp.float32),
        pltpu.VMEM((2, _BC, _N_DIMS), jnp.float32),
        pltpu.VMEM((_SEQ_LEN // 2, 128), jnp.float32),
        pltpu.VMEM((2, _N_DIMS, _BC), jnp.float32),
        pltpu.SemaphoreType.DMA,
        pltpu.SemaphoreType.DMA,
        pltpu.SemaphoreType.DMA,
        pltpu.SemaphoreType.DMA,
        pltpu.SemaphoreType.DMA,
        pltpu.SemaphoreType.DMA,
        pltpu.SemaphoreType.DMA,
        pltpu.SemaphoreType.DMA,
        pltpu.SemaphoreType.DMA,
        pltpu.SemaphoreType.DMA,
    ],
    compiler_params=pltpu.CompilerParams(use_tc_tiling_on_sc=False,
                                         needs_layout_passes=False),
)
def _embed_kernel(eidx_hbm, cidx_hbm, etab_hbm, ctab_hbm, ptab_hbm, out_hbm,
                  eidx_v, cidx_v, erows_v, crows_v, pos_v, obuf_v,
                  sem_ie0, sem_ie1, sem_ic0, sem_ic1,
                  sem_ge0, sem_ge1, sem_gc0, sem_gc1, sem_o0, sem_o1):
    sem_ie = (sem_ie0, sem_ie1)
    sem_ic = (sem_ic0, sem_ic1)
    sem_ge = (sem_ge0, sem_ge1)
    sem_gc = (sem_gc0, sem_gc1)
    sem_o = (sem_o0, sem_o1)
    wid = lax.axis_index("s") * 2 + lax.axis_index("c")
    tbase = wid * _TPW
    pltpu.sync_copy(ptab_hbm, pos_v)

    def task_sb(t):
        gt = tbase + t
        return gt // _NBC, gt % _NBC

    def idx_fetch_start(t, b):
        s, bc = task_sb(t)
        pltpu.async_copy(eidx_hbm.at[s, pl.ds(bc * _BC, _BC)],
                         eidx_v.at[b], sem_ie[b])
        pltpu.async_copy(cidx_hbm.at[s, pl.ds(bc * _BC, _BC)],
                         cidx_v.at[b], sem_ic[b])

    def idx_wait(b):
        pltpu.make_async_copy(
            eidx_hbm.at[0, pl.ds(0, _BC)], eidx_v.at[b], sem_ie[b]).wait()
        pltpu.make_async_copy(
            cidx_hbm.at[0, pl.ds(0, _BC)], cidx_v.at[b], sem_ic[b]).wait()

    def gather_start(b):
        pltpu.async_copy(etab_hbm.at[eidx_v.at[b]], erows_v.at[b], sem_ge[b])
        pltpu.async_copy(ctab_hbm.at[cidx_v.at[b]], crows_v.at[b], sem_gc[b])

    def gather_wait(b):
        pltpu.make_async_copy(
            etab_hbm.at[eidx_v.at[0]], erows_v.at[b], sem_ge[b]).wait()
        pltpu.make_async_copy(
            ctab_hbm.at[cidx_v.at[0]], crows_v.at[b], sem_gc[b]).wait()

    def out_start(t, b):
        s, bc = task_sb(t)
        pltpu.async_copy(obuf_v.at[b],
                         out_hbm.at[s, slice(None), pl.ds(bc * _BC, _BC)],
                         sem_o[b])

    def out_wait(b):
        pltpu.make_async_copy(
            obuf_v.at[b], out_hbm.at[0, slice(None), pl.ds(0, _BC)],
            sem_o[b]).wait()

    idx_fetch_start(0, 0)
    idx_fetch_start(1, 1)
    idx_wait(0)
    gather_start(0)

    _strips = tuple(
        jnp.arange(k * 16, (k + 1) * 16, dtype=jnp.int32) for k in range(8))

    def body(half, _):
        for tb in range(2):
            t = half * 2 + tb
            nb = 1 - tb

            @pl.when(t < _TPW - 1)
            def _():
                idx_wait(nb)
                gather_start(nb)

            gather_wait(tb)

            @pl.when(t < _TPW - 2)
            def _():
                idx_fetch_start(t + 2, tb)

            @pl.when(t >= 2)
            def _():
                out_wait(tb)

            s, _bc = task_sb(t)
            s2 = jnp.full((16,), s // 2, jnp.int32)
            pcol0 = (s % 2) * _N_DIMS
            erows = erows_v.at[tb]
            crows = crows_v.at[tb]

            pcv = jnp.full((16,), pcol0, jnp.int32)

            def d_body(d4, _):
                d0 = d4 * 4
                for u in range(4):
                    d = d0 + u
                    dvec = jnp.full((16,), d, jnp.int32)
                    pv = plsc.load_gather(pos_v, [s2, pcv + d])
                    for k in range(8):
                        ev = plsc.load_gather(erows, [_strips[k], dvec])
                        cv = plsc.load_gather(crows, [_strips[k], dvec])
                        obuf_v[tb, d, pl.ds(k * 16, 16)] = ev + cv + pv
                return 0

            lax.fori_loop(0, _N_DIMS // 4, d_body, 0)

            out_start(t, tb)
        return 0

    lax.fori_loop(0, _TPW // 2, body, 0)
    out_wait(0)
    out_wait(1)


def kernel(exercises, categories, exercise_embed, category_embed, position_embed):
    eidx = exercises.astype(jnp.int32).T
    cidx = categories.astype(jnp.int32).T
    pos128 = position_embed.reshape(_SEQ_LEN // 2, 128)
    out = _embed_kernel(eidx, cidx, exercise_embed, category_embed, pos128)
    return out.transpose(2, 0, 1)


# speedup vs baseline: 3.0326x; 3.0326x over previous
import functools

import jax
import jax.numpy as jnp
from jax import lax
from jax.experimental import pallas as pl
from jax.experimental.pallas import tpu as pltpu
from jax.experimental.pallas import tpu_sc as plsc

_N_DIMS = 64
_SEQ_LEN = 200
_BATCH = 4096
_ROWS = _BATCH * _SEQ_LEN
_NW = 32
_RPW = _ROWS // _NW
_CH = 128
_NCH = _RPW // _CH
_OCH = _CH // 2
_OROWS = _ROWS // 2
_PROWS = 168

_mesh = plsc.VectorSubcoreMesh(core_axis_name="c", subcore_axis_name="s")


@functools.partial(
    pl.kernel,
    mesh=_mesh,
    out_type=jax.ShapeDtypeStruct((_OROWS, 128), jnp.float32),
    scratch_types=[
        pltpu.VMEM((_NCH, _CH), jnp.int32),
        pltpu.VMEM((_NCH, _CH), jnp.int32),
        pltpu.VMEM((2, _CH, _N_DIMS), jnp.float32),
        pltpu.VMEM((2, _CH, _N_DIMS), jnp.float32),
        pltpu.VMEM((_PROWS, 128), jnp.float32),
        pltpu.VMEM((2, _OCH, 128), jnp.float32),
        pltpu.SemaphoreType.DMA,
        pltpu.SemaphoreType.DMA,
        pltpu.SemaphoreType.DMA,
        pltpu.SemaphoreType.DMA,
        pltpu.SemaphoreType.DMA,
        pltpu.SemaphoreType.DMA,
    ],
    compiler_params=pltpu.CompilerParams(use_tc_tiling_on_sc=False,
                                         disable_bounds_checks=True,
                                         disable_semaphore_checks=True),
)
def _embed_kernel(eidx_hbm, cidx_hbm, etab_hbm, ctab_hbm, ptab_hbm, out_hbm,
                  eidx_v, cidx_v, erows_v, crows_v, pos_v, obuf_v,
                  sem_ge0, sem_ge1, sem_gc0, sem_gc1, sem_o0, sem_o1):
    sem_ge = (sem_ge0, sem_ge1)
    sem_gc = (sem_gc0, sem_gc1)
    sem_o = (sem_o0, sem_o1)
    wid = lax.axis_index("s") * 2 + lax.axis_index("c")
    base = wid * _RPW
    obase = wid * (_RPW // 2)
    pltpu.sync_copy(ptab_hbm, pos_v.at[pl.ds(0, _SEQ_LEN // 2)])
    pltpu.sync_copy(ptab_hbm.at[pl.ds(0, _PROWS - _SEQ_LEN // 2)],
                    pos_v.at[pl.ds(_SEQ_LEN // 2, _PROWS - _SEQ_LEN // 2)])
    pltpu.sync_copy(eidx_hbm.at[pl.ds(wid * _NCH, _NCH)], eidx_v)
    pltpu.sync_copy(cidx_hbm.at[pl.ds(wid * _NCH, _NCH)], cidx_v)

    def gather_start(c, b):
        pltpu.async_copy(etab_hbm.at[eidx_v.at[c]], erows_v.at[b], sem_ge[b])
        pltpu.async_copy(ctab_hbm.at[cidx_v.at[c]], crows_v.at[b], sem_gc[b])

    def gather_wait(b):
        pltpu.make_async_copy(
            etab_hbm.at[eidx_v.at[0]], erows_v.at[b], sem_ge[b]).wait()
        pltpu.make_async_copy(
            ctab_hbm.at[cidx_v.at[0]], crows_v.at[b], sem_gc[b]).wait()

    def out_wait(b):
        pltpu.make_async_copy(
            obuf_v.at[b], out_hbm.at[pl.ds(0, _OCH)], sem_o[b]).wait()

    gather_start(0, 0)

    def outer(half, _):
        cb = half * 2
        for b in range(2):
            c = cb + b
            nb = 1 - b

            @pl.when(c < _NCH - 1)
            def _():
                gather_start(c + 1, nb)

            gather_wait(b)

            @pl.when(c >= 2)
            def _():
                out_wait(b)

            pb2 = ((c * _CH) % _SEQ_LEN) // 2

            @plsc.parallel_loop(0, _OCH, unroll=8)
            def row_body(q):
                prow = pb2 + q
                for h in range(8):
                    r = 2 * q + h // 4
                    g = (h % 4) * 16
                    oslice = pl.ds(h * 16, 16)
                    obuf_v[b, q, oslice] = (erows_v[b, r, pl.ds(g, 16)]
                                            + crows_v[b, r, pl.ds(g, 16)]
                                            + pos_v[prow, oslice])
            pltpu.async_copy(
                obuf_v.at[b], out_hbm.at[pl.ds(obase + c * _OCH, _OCH)],
                sem_o[b])
        return 0

    lax.fori_loop(0, _NCH // 2, outer, 0)
    out_wait(0)
    out_wait(1)


def kernel(exercises, categories, exercise_embed, category_embed, position_embed):
    eidx = exercises.reshape(_ROWS // _CH, _CH).astype(jnp.int32)
    cidx = categories.reshape(_ROWS // _CH, _CH).astype(jnp.int32)
    pos128 = position_embed.reshape(_SEQ_LEN // 2, 128)
    out = _embed_kernel(eidx, cidx, exercise_embed, category_embed, pos128)
    return out.reshape(_BATCH, _SEQ_LEN, _N_DIMS)
